# Initial kernel scaffold; baseline (speedup 1.0000x reference)
import jax
import jax.numpy as jnp
from jax.experimental import pallas as pl


def kernel(x, edge_attr, edge_index, batch, Wn, bn, We, be, W1s, b1s, W2s, b2s, eps, Wm1, bm1, Wm2, bm2):
    raise NotImplementedError("write your pallas kernel here")


# speedup vs baseline: 2.5591x; 2.5591x over previous
import functools

import jax
import jax.numpy as jnp
from jax import lax
from jax.experimental import pallas as pl
from jax.experimental.pallas import tpu as pltpu
from jax.experimental.pallas import tpu_sc as plsc

N = 10000
E = 320000
H = 128
G = 64

NC = 2
NS = 16
NW = NC * NS
EPW = E // NW
CHUNK = 80
NCHUNK_W = EPW // CHUNK
NCHUNK_N = N // CHUNK


def _edge_sc_body(h_hbm, e_hbm, src_hbm, dst_hbm, out_hbm,
                  sidx, didx, hrows, erows, zbuf, aggr, sem):
    cid = lax.axis_index("c")
    sid = lax.axis_index("s")
    wid = cid * NS + sid
    zero16 = jnp.zeros((16,), jnp.float32)

    def zfill(i, c):
        for j in range(8):
            zbuf[i, pl.ds(j * 16, 16)] = zero16
        return c
    lax.fori_loop(0, CHUNK, zfill, 0)

    def zcopy(m, c):
        k = sid + m * NS
        @pl.when(k < NCHUNK_N)
        def _():
            pltpu.sync_copy(zbuf, aggr.at[pl.ds(k * CHUNK, CHUNK)])
        return c
    lax.fori_loop(0, (NCHUNK_N + NS - 1) // NS, zcopy, 0)
    plsc.subcore_barrier()

    base0 = wid * EPW

    def chunk(kk, c):
        base = base0 + kk * CHUNK
        pltpu.sync_copy(src_hbm.at[pl.ds(base, CHUNK)], sidx)
        pltpu.sync_copy(dst_hbm.at[pl.ds(base, CHUNK)], didx)
        pltpu.async_copy(h_hbm.at[sidx], hrows, sem).wait()
        pltpu.sync_copy(e_hbm.at[pl.ds(base, CHUNK)], erows)

        def compute(i, c2):
            for j in range(8):
                sl = pl.ds(j * 16, 16)
                erows[i, sl] = jnp.maximum(erows[i, sl] + hrows[i, sl], 0.0)
            return c2
        lax.fori_loop(0, CHUNK, compute, 0)

        pltpu.sync_copy(erows, aggr.at[didx], add=True)
        return c
    lax.fori_loop(0, NCHUNK_W, chunk, 0)

    plsc.subcore_barrier()

    def ocopy(m, c):
        k = sid + m * NS
        @pl.when(k < NCHUNK_N)
        def _():
            pltpu.sync_copy(aggr.at[pl.ds(k * CHUNK, CHUNK)],
                            out_hbm.at[pl.ds(cid * N + k * CHUNK, CHUNK)])
        return c
    lax.fori_loop(0, (NCHUNK_N + NS - 1) // NS, ocopy, 0)


_edge_aggregate_fn = None


def _edge_aggregate(h, e, src, dst):
    global _edge_aggregate_fn
    if _edge_aggregate_fn is None:
        _edge_aggregate_fn = functools.partial(
            pl.kernel,
            mesh=plsc.VectorSubcoreMesh(core_axis_name="c",
                                        subcore_axis_name="s"),
            out_type=jax.ShapeDtypeStruct((NC * N, H), jnp.float32),
            scratch_types=[
                pltpu.VMEM((CHUNK,), jnp.int32),
                pltpu.VMEM((CHUNK,), jnp.int32),
                pltpu.VMEM((CHUNK, H), jnp.float32),
                pltpu.VMEM((CHUNK, H), jnp.float32),
                pltpu.VMEM((CHUNK, H), jnp.float32),
                pltpu.VMEM_SHARED((N, H), jnp.float32),
                pltpu.SemaphoreType.DMA,
            ],
        )(_edge_sc_body)
    return _edge_aggregate_fn(h, e, src, dst)


def _embed(a, w, b_row, blk):
    n, d = a.shape
    h = w.shape[1]

    def body(a_ref, w_ref, b_ref, o_ref):
        o_ref[...] = (jnp.dot(a_ref[...], w_ref[...],
                              preferred_element_type=jnp.float32,
                             precision=lax.Precision.HIGHEST)
                      + b_ref[...])

    return pl.pallas_call(
        body,
        grid=(n // blk,),
        in_specs=[pl.BlockSpec((blk, d), lambda i: (i, 0)),
                  pl.BlockSpec((d, h), lambda i: (0, 0)),
                  pl.BlockSpec((1, h), lambda i: (0, 0))],
        out_specs=pl.BlockSpec((blk, h), lambda i: (i, 0)),
        out_shape=jax.ShapeDtypeStruct((n, h), jnp.float32),
    )(a, w, b_row)


def _layer_update(h, a0, a1, w1, b1_row, w2, b2_row, scale_row, blk=2000):
    n, hd = h.shape
    h2 = w1.shape[1]

    def body(h_ref, a0_ref, a1_ref, w1_ref, b1_ref, w2_ref, b2_ref, s_ref,
             o_ref):
        z = h_ref[...] * s_ref[...] + a0_ref[...] + a1_ref[...]
        z1 = jnp.maximum(jnp.dot(z, w1_ref[...],
                                 preferred_element_type=jnp.float32,
                             precision=lax.Precision.HIGHEST)
                         + b1_ref[...], 0.0)
        o_ref[...] = jnp.maximum(jnp.dot(z1, w2_ref[...],
                                         preferred_element_type=jnp.float32,
                             precision=lax.Precision.HIGHEST)
                                 + b2_ref[...], 0.0)

    return pl.pallas_call(
        body,
        grid=(n // blk,),
        in_specs=[pl.BlockSpec((blk, hd), lambda i: (i, 0)),
                  pl.BlockSpec((blk, hd), lambda i: (i, 0)),
                  pl.BlockSpec((blk, hd), lambda i: (i, 0)),
                  pl.BlockSpec((hd, h2), lambda i: (0, 0)),
                  pl.BlockSpec((1, h2), lambda i: (0, 0)),
                  pl.BlockSpec((h2, hd), lambda i: (0, 0)),
                  pl.BlockSpec((1, hd), lambda i: (0, 0)),
                  pl.BlockSpec((1, hd), lambda i: (0, 0))],
        out_specs=pl.BlockSpec((blk, hd), lambda i: (i, 0)),
        out_shape=jax.ShapeDtypeStruct((n, hd), jnp.float32),
    )(h, a0, a1, w1, b1_row, w2, b2_row, scale_row)


def _pool_mlp(h, batch3d, wm1, bm1_row, wm2_row, bm2_11, blk=2000):
    n, hd = h.shape
    steps = n // blk
    hh = wm1.shape[1]

    def body(h_ref, b_ref, w1_ref, b1_ref, w2_ref, b2_ref, o_ref,
             acc_ref, cnt_ref):
        g = pl.program_id(0)

        @pl.when(g == 0)
        def _():
            acc_ref[...] = jnp.zeros_like(acc_ref)
            cnt_ref[...] = jnp.zeros_like(cnt_ref)

        ids = b_ref[0, :, :]
        iota = lax.broadcasted_iota(jnp.int32, (G, blk), 0)
        onehot = (iota == ids).astype(jnp.float32)
        acc_ref[...] += jnp.dot(onehot, h_ref[...],
                                preferred_element_type=jnp.float32,
                             precision=lax.Precision.HIGHEST)
        cnt_ref[...] += jnp.broadcast_to(
            jnp.sum(onehot, axis=1, keepdims=True), cnt_ref.shape)

        @pl.when(g == steps - 1)
        def _():
            pooled = acc_ref[...] / jnp.maximum(cnt_ref[...], 1.0)
            t = jnp.maximum(jnp.dot(pooled, w1_ref[...],
                                    preferred_element_type=jnp.float32,
                             precision=lax.Precision.HIGHEST)
                            + b1_ref[...], 0.0)
            o_ref[...] = (jnp.sum(t * w2_ref[...], axis=1, keepdims=True)
                          + b2_ref[...])

    return pl.pallas_call(
        body,
        grid=(steps,),
        in_specs=[pl.BlockSpec((blk, hd), lambda i: (i, 0)),
                  pl.BlockSpec((1, 1, blk), lambda i: (i, 0, 0)),
                  pl.BlockSpec((hd, hh), lambda i: (0, 0)),
                  pl.BlockSpec((1, hh), lambda i: (0, 0)),
                  pl.BlockSpec((1, hh), lambda i: (0, 0)),
                  pl.BlockSpec((1, 1), lambda i: (0, 0))],
        out_specs=pl.BlockSpec((G, 1), lambda i: (0, 0)),
        out_shape=jax.ShapeDtypeStruct((G, 1), jnp.float32),
        scratch_shapes=[pltpu.VMEM((G, hd), jnp.float32),
                        pltpu.VMEM((G, hd), jnp.float32)],
    )(h, batch3d, wm1, bm1_row, wm2_row, bm2_11)


def kernel(x, edge_attr, edge_index, batch, Wn, bn, We, be,
                 W1s, b1s, W2s, b2s, eps, Wm1, bm1, Wm2, bm2):
    src = edge_index[0]
    dst = edge_index[1]
    h = _embed(x, Wn, bn.reshape(1, -1), blk=2000)
    e = _embed(edge_attr, We, be.reshape(1, -1), blk=4000)
    for i in range(W1s.shape[0]):
        agg = _edge_aggregate(h, e, src, dst)
        scale_row = jnp.broadcast_to(1.0 + eps[i], (1, H))
        h = _layer_update(h, agg[:N], agg[N:], W1s[i], b1s[i].reshape(1, -1),
                          W2s[i], b2s[i].reshape(1, -1), scale_row)
    out = _pool_mlp(h, batch.reshape(N // 2000, 1, 2000), Wm1,
                    bm1.reshape(1, -1), Wm2.reshape(1, -1),
                    bm2.reshape(1, 1))
    return out


# speedup vs baseline: 4.7183x; 1.8437x over previous
import functools

import jax
import jax.numpy as jnp
from jax import lax
from jax.experimental import pallas as pl
from jax.experimental.pallas import tpu as pltpu
from jax.experimental.pallas import tpu_sc as plsc

N = 10000
E = 320000
H = 128
G = 64

NC = 2
NS = 16
NW = NC * NS
EPW = E // NW
CHUNK = 80
NCHUNK_W = EPW // CHUNK
NCHUNK_N = N // CHUNK


def _edge_sc_body(h_hbm, e_hbm, src_hbm, dst_hbm, out_hbm,
                  sidx0, sidx1, didx0, didx1,
                  hrows0, hrows1, erows0, erows1, aggr,
                  gsem0, gsem1, esem0, esem1, isem0, isem1):
    cid = lax.axis_index("c")
    sid = lax.axis_index("s")
    wid = cid * NS + sid
    zero16 = jnp.zeros((16,), jnp.float32)
    sidx = (sidx0, sidx1)
    didx = (didx0, didx1)
    hrows = (hrows0, hrows1)
    erows = (erows0, erows1)
    gsem = (gsem0, gsem1)
    esem = (esem0, esem1)
    isem = (isem0, isem1)
    base0 = wid * EPW

    def zfill(i, c):
        for j in range(8):
            erows0[i, pl.ds(j * 16, 16)] = zero16
        return c
    lax.fori_loop(0, CHUNK, zfill, 0)

    def zcopy(m, c):
        k = sid + m * NS
        @pl.when(k < NCHUNK_N)
        def _():
            pltpu.sync_copy(erows0, aggr.at[pl.ds(k * CHUNK, CHUNK)])
        return c
    lax.fori_loop(0, (NCHUNK_N + NS - 1) // NS, zcopy, 0)
    plsc.subcore_barrier()

    def issue_idx(k, b):
        pltpu.make_async_copy(src_hbm.at[pl.ds(base0 + k * CHUNK, CHUNK)],
                              sidx[b], isem[b]).start()
        pltpu.make_async_copy(dst_hbm.at[pl.ds(base0 + k * CHUNK, CHUNK)],
                              didx[b], isem[b]).start()

    def wait_idx(k, b):
        pltpu.make_async_copy(src_hbm.at[pl.ds(base0 + k * CHUNK, CHUNK)],
                              sidx[b], isem[b]).wait()
        pltpu.make_async_copy(dst_hbm.at[pl.ds(base0 + k * CHUNK, CHUNK)],
                              didx[b], isem[b]).wait()

    def issue_data(k, b):
        pltpu.make_async_copy(h_hbm.at[sidx[b]], hrows[b], gsem[b]).start()
        pltpu.make_async_copy(e_hbm.at[pl.ds(base0 + k * CHUNK, CHUNK)],
                              erows[b], esem[b]).start()

    def wait_data(k, b):
        pltpu.make_async_copy(h_hbm.at[sidx[b]], hrows[b], gsem[b]).wait()
        pltpu.make_async_copy(e_hbm.at[pl.ds(base0 + k * CHUNK, CHUNK)],
                              erows[b], esem[b]).wait()

    def work(k, b):
        def compute(i, c2):
            for j in range(8):
                sl = pl.ds(j * 16, 16)
                erows[b][i, sl] = jnp.maximum(
                    erows[b][i, sl] + hrows[b][i, sl], 0.0)
            return c2
        lax.fori_loop(0, CHUNK, compute, 0)
        pltpu.sync_copy(erows[b], aggr.at[didx[b]], add=True)

    issue_idx(0, 0)
    wait_idx(0, 0)
    issue_data(0, 0)
    issue_idx(1, 1)

    def step(k, b):
        wait_data(k, b)
        wait_idx(k + 1, 1 - b)
        issue_data(k + 1, 1 - b)
        work(k, b)
        @pl.when(k + 2 < NCHUNK_W)
        def _():
            issue_idx(k + 2, b)

    def pair(p, c):
        k = 2 * p
        for b in range(2):
            step(k + b, b)
        return c
    lax.fori_loop(0, (NCHUNK_W - 1) // 2, pair, 0)
    wait_data(NCHUNK_W - 1, 0)
    work(NCHUNK_W - 1, 0)

    plsc.subcore_barrier()

    def ocopy(m, c):
        k = sid + m * NS
        @pl.when(k < NCHUNK_N)
        def _():
            pltpu.sync_copy(aggr.at[pl.ds(k * CHUNK, CHUNK)],
                            out_hbm.at[pl.ds(cid * N + k * CHUNK, CHUNK)])
        return c
    lax.fori_loop(0, (NCHUNK_N + NS - 1) // NS, ocopy, 0)


_edge_aggregate_fn = None


def _edge_aggregate(h, e, src, dst):
    global _edge_aggregate_fn
    if _edge_aggregate_fn is None:
        _edge_aggregate_fn = functools.partial(
            pl.kernel,
            mesh=plsc.VectorSubcoreMesh(core_axis_name="c",
                                        subcore_axis_name="s"),
            out_type=jax.ShapeDtypeStruct((NC * N, H), jnp.float32),
            scratch_types=[
                pltpu.VMEM((CHUNK,), jnp.int32),
                pltpu.VMEM((CHUNK,), jnp.int32),
                pltpu.VMEM((CHUNK,), jnp.int32),
                pltpu.VMEM((CHUNK,), jnp.int32),
                pltpu.VMEM((CHUNK, H), jnp.float32),
                pltpu.VMEM((CHUNK, H), jnp.float32),
                pltpu.VMEM((CHUNK, H), jnp.float32),
                pltpu.VMEM((CHUNK, H), jnp.float32),
                pltpu.VMEM_SHARED((N, H), jnp.float32),
                pltpu.SemaphoreType.DMA,
                pltpu.SemaphoreType.DMA,
                pltpu.SemaphoreType.DMA,
                pltpu.SemaphoreType.DMA,
                pltpu.SemaphoreType.DMA,
                pltpu.SemaphoreType.DMA,
            ],
        )(_edge_sc_body)
    return _edge_aggregate_fn(h, e, src, dst)


def _embed(a, w, b_row, blk):
    n, d = a.shape
    h = w.shape[1]

    def body(a_ref, w_ref, b_ref, o_ref):
        o_ref[...] = (jnp.dot(a_ref[...], w_ref[...],
                              preferred_element_type=jnp.float32,
                             precision=lax.Precision.HIGHEST)
                      + b_ref[...])

    return pl.pallas_call(
        body,
        grid=(n // blk,),
        in_specs=[pl.BlockSpec((blk, d), lambda i: (i, 0)),
                  pl.BlockSpec((d, h), lambda i: (0, 0)),
                  pl.BlockSpec((1, h), lambda i: (0, 0))],
        out_specs=pl.BlockSpec((blk, h), lambda i: (i, 0)),
        out_shape=jax.ShapeDtypeStruct((n, h), jnp.float32),
    )(a, w, b_row)


def _layer_update(h, a0, a1, w1, b1_row, w2, b2_row, scale_row, blk=2000):
    n, hd = h.shape
    h2 = w1.shape[1]

    def body(h_ref, a0_ref, a1_ref, w1_ref, b1_ref, w2_ref, b2_ref, s_ref,
             o_ref):
        z = h_ref[...] * s_ref[...] + a0_ref[...] + a1_ref[...]
        z1 = jnp.maximum(jnp.dot(z, w1_ref[...],
                                 preferred_element_type=jnp.float32,
                             precision=lax.Precision.HIGHEST)
                         + b1_ref[...], 0.0)
        o_ref[...] = jnp.maximum(jnp.dot(z1, w2_ref[...],
                                         preferred_element_type=jnp.float32,
                             precision=lax.Precision.HIGHEST)
                                 + b2_ref[...], 0.0)

    return pl.pallas_call(
        body,
        grid=(n // blk,),
        in_specs=[pl.BlockSpec((blk, hd), lambda i: (i, 0)),
                  pl.BlockSpec((blk, hd), lambda i: (i, 0)),
                  pl.BlockSpec((blk, hd), lambda i: (i, 0)),
                  pl.BlockSpec((hd, h2), lambda i: (0, 0)),
                  pl.BlockSpec((1, h2), lambda i: (0, 0)),
                  pl.BlockSpec((h2, hd), lambda i: (0, 0)),
                  pl.BlockSpec((1, hd), lambda i: (0, 0)),
                  pl.BlockSpec((1, hd), lambda i: (0, 0))],
        out_specs=pl.BlockSpec((blk, hd), lambda i: (i, 0)),
        out_shape=jax.ShapeDtypeStruct((n, hd), jnp.float32),
    )(h, a0, a1, w1, b1_row, w2, b2_row, scale_row)


def _pool_mlp(h, batch3d, wm1, bm1_row, wm2_row, bm2_11, blk=2000):
    n, hd = h.shape
    steps = n // blk
    hh = wm1.shape[1]

    def body(h_ref, b_ref, w1_ref, b1_ref, w2_ref, b2_ref, o_ref,
             acc_ref, cnt_ref):
        g = pl.program_id(0)

        @pl.when(g == 0)
        def _():
            acc_ref[...] = jnp.zeros_like(acc_ref)
            cnt_ref[...] = jnp.zeros_like(cnt_ref)

        ids = b_ref[0, :, :]
        iota = lax.broadcasted_iota(jnp.int32, (G, blk), 0)
        onehot = (iota == ids).astype(jnp.float32)
        acc_ref[...] += jnp.dot(onehot, h_ref[...],
                                preferred_element_type=jnp.float32,
                             precision=lax.Precision.HIGHEST)
        cnt_ref[...] += jnp.broadcast_to(
            jnp.sum(onehot, axis=1, keepdims=True), cnt_ref.shape)

        @pl.when(g == steps - 1)
        def _():
            pooled = acc_ref[...] / jnp.maximum(cnt_ref[...], 1.0)
            t = jnp.maximum(jnp.dot(pooled, w1_ref[...],
                                    preferred_element_type=jnp.float32,
                             precision=lax.Precision.HIGHEST)
                            + b1_ref[...], 0.0)
            o_ref[...] = (jnp.sum(t * w2_ref[...], axis=1, keepdims=True)
                          + b2_ref[...])

    return pl.pallas_call(
        body,
        grid=(steps,),
        in_specs=[pl.BlockSpec((blk, hd), lambda i: (i, 0)),
                  pl.BlockSpec((1, 1, blk), lambda i: (i, 0, 0)),
                  pl.BlockSpec((hd, hh), lambda i: (0, 0)),
                  pl.BlockSpec((1, hh), lambda i: (0, 0)),
                  pl.BlockSpec((1, hh), lambda i: (0, 0)),
                  pl.BlockSpec((1, 1), lambda i: (0, 0))],
        out_specs=pl.BlockSpec((G, 1), lambda i: (0, 0)),
        out_shape=jax.ShapeDtypeStruct((G, 1), jnp.float32),
        scratch_shapes=[pltpu.VMEM((G, hd), jnp.float32),
                        pltpu.VMEM((G, hd), jnp.float32)],
    )(h, batch3d, wm1, bm1_row, wm2_row, bm2_11)


def kernel(x, edge_attr, edge_index, batch, Wn, bn, We, be,
                 W1s, b1s, W2s, b2s, eps, Wm1, bm1, Wm2, bm2):
    src = edge_index[0]
    dst = edge_index[1]
    h = _embed(x, Wn, bn.reshape(1, -1), blk=2000)
    e = _embed(edge_attr, We, be.reshape(1, -1), blk=4000)
    for i in range(W1s.shape[0]):
        agg = _edge_aggregate(h, e, src, dst)
        scale_row = jnp.broadcast_to(1.0 + eps[i], (1, H))
        h = _layer_update(h, agg[:N], agg[N:], W1s[i], b1s[i].reshape(1, -1),
                          W2s[i], b2s[i].reshape(1, -1), scale_row)
    out = _pool_mlp(h, batch.reshape(N // 2000, 1, 2000), Wm1,
                    bm1.reshape(1, -1), Wm2.reshape(1, -1),
                    bm2.reshape(1, 1))
    return out


# speedup vs baseline: 5.5487x; 1.1760x over previous
import functools

import jax
import jax.numpy as jnp
from jax import lax
from jax.experimental import pallas as pl
from jax.experimental.pallas import tpu as pltpu
from jax.experimental.pallas import tpu_sc as plsc

N = 10000
E = 320000
H = 128
G = 64

NC = 2
NS = 16
NW = NC * NS
EPW = E // NW
CHUNK = 80
NCHUNK_W = EPW // CHUNK
NCHUNK_N = N // CHUNK


def _edge_sc_body(h_hbm, e_hbm, src_hbm, dst_hbm, out_hbm,
                  sidx0, sidx1, didx0, didx1, didx2,
                  hrows0, hrows1, erows0, erows1, aggr,
                  gsem0, gsem1, esem0, esem1, isem0, isem1, ssem0, ssem1):
    cid = lax.axis_index("c")
    sid = lax.axis_index("s")
    wid = cid * NS + sid
    zero16 = jnp.zeros((16,), jnp.float32)
    sidx = (sidx0, sidx1)
    didx = (didx0, didx1, didx2)
    hrows = (hrows0, hrows1)
    erows = (erows0, erows1)
    gsem = (gsem0, gsem1)
    esem = (esem0, esem1)
    isem = (isem0, isem1)
    ssem = (ssem0, ssem1)
    base0 = wid * EPW

    def zfill(i, c):
        for j in range(8):
            erows0[i, pl.ds(j * 16, 16)] = zero16
        return c
    lax.fori_loop(0, CHUNK, zfill, 0)

    def zcopy(m, c):
        k = sid + m * NS
        @pl.when(k < NCHUNK_N)
        def _():
            pltpu.sync_copy(erows0, aggr.at[pl.ds(k * CHUNK, CHUNK)])
        return c
    lax.fori_loop(0, (NCHUNK_N + NS - 1) // NS, zcopy, 0)
    plsc.subcore_barrier()

    def issue_idx(k, eb, db):
        pltpu.make_async_copy(src_hbm.at[pl.ds(base0 + k * CHUNK, CHUNK)],
                              sidx[eb], isem[eb]).start()
        pltpu.make_async_copy(dst_hbm.at[pl.ds(base0 + k * CHUNK, CHUNK)],
                              didx[db], isem[eb]).start()

    def wait_idx(k, eb, db):
        pltpu.make_async_copy(src_hbm.at[pl.ds(base0 + k * CHUNK, CHUNK)],
                              sidx[eb], isem[eb]).wait()
        pltpu.make_async_copy(dst_hbm.at[pl.ds(base0 + k * CHUNK, CHUNK)],
                              didx[db], isem[eb]).wait()

    def issue_data(k, eb):
        pltpu.make_async_copy(h_hbm.at[sidx[eb]], hrows[eb], gsem[eb]).start()
        pltpu.make_async_copy(e_hbm.at[pl.ds(base0 + k * CHUNK, CHUNK)],
                              erows[eb], esem[eb]).start()

    def wait_data(k, eb):
        pltpu.make_async_copy(h_hbm.at[sidx[eb]], hrows[eb], gsem[eb]).wait()
        pltpu.make_async_copy(e_hbm.at[pl.ds(base0 + k * CHUNK, CHUNK)],
                              erows[eb], esem[eb]).wait()

    def compute(eb):
        def body(i, c2):
            for r in range(2):
                for j in range(8):
                    sl = pl.ds(j * 16, 16)
                    erows[eb][2 * i + r, sl] = jnp.maximum(
                        erows[eb][2 * i + r, sl] + hrows[eb][2 * i + r, sl],
                        0.0)
            return c2
        lax.fori_loop(0, CHUNK // 2, body, 0)

    def start_scatter(eb, db):
        pltpu.make_async_copy(erows[eb], aggr.at[didx[db]],
                              ssem[eb]).start(add=True)

    def wait_scatter(eb, db):
        pltpu.make_async_copy(erows[eb], aggr.at[didx[db]],
                              ssem[eb]).wait()

    issue_idx(0, 0, 0)
    wait_idx(0, 0, 0)
    issue_data(0, 0)
    issue_idx(1, 1, 1)
    wait_data(0, 0)
    wait_idx(1, 1, 1)
    issue_data(1, 1)
    compute(0)
    start_scatter(0, 0)
    issue_idx(2, 0, 2)

    def step(k, km, guard_tail=False):
        eb, db = km % 2, km % 3
        wait_data(k, eb)
        if not guard_tail or km + 1 < NCHUNK_W:
            wait_idx(k + 1, 1 - eb, (km + 1) % 3)
        wait_scatter(1 - eb, (km - 1) % 3)
        if not guard_tail or km + 1 < NCHUNK_W:
            issue_data(k + 1, 1 - eb)
        compute(eb)
        start_scatter(eb, db)
        if not guard_tail or km + 2 < NCHUNK_W:
            issue_idx(k + 2, eb, (km + 2) % 3)

    def six(p, c):
        for u in range(6):
            step(6 * p + u + 1, u + 1)
        return c
    lax.fori_loop(0, (NCHUNK_W - 5) // 6, six, 0)
    for k in range(NCHUNK_W - 4, NCHUNK_W):
        step(k, k, guard_tail=True)
    wait_scatter((NCHUNK_W - 1) % 2, (NCHUNK_W - 1) % 3)

    plsc.subcore_barrier()

    def ocopy(m, c):
        k = sid + m * NS
        @pl.when(k < NCHUNK_N)
        def _():
            pltpu.sync_copy(aggr.at[pl.ds(k * CHUNK, CHUNK)],
                            out_hbm.at[pl.ds(cid * N + k * CHUNK, CHUNK)])
        return c
    lax.fori_loop(0, (NCHUNK_N + NS - 1) // NS, ocopy, 0)


_edge_aggregate_fn = None


def _edge_aggregate(h, e, src, dst):
    global _edge_aggregate_fn
    if _edge_aggregate_fn is None:
        _edge_aggregate_fn = functools.partial(
            pl.kernel,
            mesh=plsc.VectorSubcoreMesh(core_axis_name="c",
                                        subcore_axis_name="s"),
            out_type=jax.ShapeDtypeStruct((NC * N, H), jnp.float32),
            scratch_types=(
                [pltpu.VMEM((CHUNK,), jnp.int32)] * 5
                + [pltpu.VMEM((CHUNK, H), jnp.float32)] * 4
                + [pltpu.VMEM_SHARED((N, H), jnp.float32)]
                + [pltpu.SemaphoreType.DMA] * 8
            ),
        )(_edge_sc_body)
    return _edge_aggregate_fn(h, e, src, dst)


def _embed(a, w, b_row, blk):
    n, d = a.shape
    h = w.shape[1]

    def body(a_ref, w_ref, b_ref, o_ref):
        o_ref[...] = (jnp.dot(a_ref[...], w_ref[...],
                              preferred_element_type=jnp.float32,
                             precision=lax.Precision.HIGHEST)
                      + b_ref[...])

    return pl.pallas_call(
        body,
        grid=(n // blk,),
        in_specs=[pl.BlockSpec((blk, d), lambda i: (i, 0)),
                  pl.BlockSpec((d, h), lambda i: (0, 0)),
                  pl.BlockSpec((1, h), lambda i: (0, 0))],
        out_specs=pl.BlockSpec((blk, h), lambda i: (i, 0)),
        out_shape=jax.ShapeDtypeStruct((n, h), jnp.float32),
    )(a, w, b_row)


def _embed_edges(a, w, b_row, blk=8000):
    n, d = a.shape
    h = w.shape[1]

    def body(a_ref, w_ref, b_ref, o_ref):
        af = a_ref[...]
        wf = w_ref[...]
        ah = af.astype(jnp.bfloat16)
        al = (af - ah.astype(jnp.float32)).astype(jnp.bfloat16)
        wh = wf.astype(jnp.bfloat16)
        wl = (wf - wh.astype(jnp.float32)).astype(jnp.bfloat16)
        o = (jnp.dot(ah, wh, preferred_element_type=jnp.float32)
             + jnp.dot(ah, wl, preferred_element_type=jnp.float32)
             + jnp.dot(al, wh, preferred_element_type=jnp.float32))
        o_ref[...] = o + b_ref[...]

    return pl.pallas_call(
        body,
        grid=(n // blk,),
        in_specs=[pl.BlockSpec((blk, d), lambda i: (i, 0)),
                  pl.BlockSpec((d, h), lambda i: (0, 0)),
                  pl.BlockSpec((1, h), lambda i: (0, 0))],
        out_specs=pl.BlockSpec((blk, h), lambda i: (i, 0)),
        out_shape=jax.ShapeDtypeStruct((n, h), jnp.float32),
    )(a, w, b_row)


def _dot3(a, w):
    ah = a.astype(jnp.bfloat16)
    al = (a - ah.astype(jnp.float32)).astype(jnp.bfloat16)
    wh = w.astype(jnp.bfloat16)
    wl = (w - wh.astype(jnp.float32)).astype(jnp.bfloat16)
    return (jnp.dot(ah, wh, preferred_element_type=jnp.float32)
            + jnp.dot(ah, wl, preferred_element_type=jnp.float32)
            + jnp.dot(al, wh, preferred_element_type=jnp.float32))


def _layer_update(h, agg, w1, b1_row, w2, b2_row, scale_row, blk=2000):
    n, hd = h.shape
    h2 = w1.shape[1]
    nblk = n // blk

    def body(h_ref, a0_ref, a1_ref, w1_ref, b1_ref, w2_ref, b2_ref, s_ref,
             o_ref):
        z = h_ref[...] * s_ref[...] + a0_ref[...] + a1_ref[...]
        z1 = jnp.maximum(_dot3(z, w1_ref[...]) + b1_ref[...], 0.0)
        o_ref[...] = jnp.maximum(_dot3(z1, w2_ref[...]) + b2_ref[...], 0.0)

    return pl.pallas_call(
        body,
        grid=(nblk,),
        in_specs=[pl.BlockSpec((blk, hd), lambda i: (i, 0)),
                  pl.BlockSpec((blk, hd), lambda i: (i, 0)),
                  pl.BlockSpec((blk, hd), lambda i, nb=nblk: (i + nb, 0)),
                  pl.BlockSpec((hd, h2), lambda i: (0, 0)),
                  pl.BlockSpec((1, h2), lambda i: (0, 0)),
                  pl.BlockSpec((h2, hd), lambda i: (0, 0)),
                  pl.BlockSpec((1, hd), lambda i: (0, 0)),
                  pl.BlockSpec((1, hd), lambda i: (0, 0))],
        out_specs=pl.BlockSpec((blk, hd), lambda i: (i, 0)),
        out_shape=jax.ShapeDtypeStruct((n, hd), jnp.float32),
    )(h, agg, agg, w1, b1_row, w2, b2_row, scale_row)


def _pool_mlp(h, batch3d, wm1, bm1_row, wm2_row, bm2_11, blk=2000):
    n, hd = h.shape
    steps = n // blk
    hh = wm1.shape[1]

    def body(h_ref, b_ref, w1_ref, b1_ref, w2_ref, b2_ref, o_ref,
             acc_ref, cnt_ref):
        g = pl.program_id(0)

        @pl.when(g == 0)
        def _():
            acc_ref[...] = jnp.zeros_like(acc_ref)
            cnt_ref[...] = jnp.zeros_like(cnt_ref)

        ids = b_ref[0, :, :]
        iota = lax.broadcasted_iota(jnp.int32, (G, blk), 0)
        onehot = (iota == ids).astype(jnp.float32)
        acc_ref[...] += jnp.dot(onehot, h_ref[...],
                                preferred_element_type=jnp.float32,
                             precision=lax.Precision.HIGHEST)
        cnt_ref[...] += jnp.broadcast_to(
            jnp.sum(onehot, axis=1, keepdims=True), cnt_ref.shape)

        @pl.when(g == steps - 1)
        def _():
            pooled = acc_ref[...] / jnp.maximum(cnt_ref[...], 1.0)
            t = jnp.maximum(jnp.dot(pooled, w1_ref[...],
                                    preferred_element_type=jnp.float32,
                             precision=lax.Precision.HIGHEST)
                            + b1_ref[...], 0.0)
            o_ref[...] = (jnp.sum(t * w2_ref[...], axis=1, keepdims=True)
                          + b2_ref[...])

    return pl.pallas_call(
        body,
        grid=(steps,),
        in_specs=[pl.BlockSpec((blk, hd), lambda i: (i, 0)),
                  pl.BlockSpec((1, 1, blk), lambda i: (i, 0, 0)),
                  pl.BlockSpec((hd, hh), lambda i: (0, 0)),
                  pl.BlockSpec((1, hh), lambda i: (0, 0)),
                  pl.BlockSpec((1, hh), lambda i: (0, 0)),
                  pl.BlockSpec((1, 1), lambda i: (0, 0))],
        out_specs=pl.BlockSpec((G, 1), lambda i: (0, 0)),
        out_shape=jax.ShapeDtypeStruct((G, 1), jnp.float32),
        scratch_shapes=[pltpu.VMEM((G, hd), jnp.float32),
                        pltpu.VMEM((G, hd), jnp.float32)],
    )(h, batch3d, wm1, bm1_row, wm2_row, bm2_11)


def kernel(x, edge_attr, edge_index, batch, Wn, bn, We, be,
                 W1s, b1s, W2s, b2s, eps, Wm1, bm1, Wm2, bm2):
    src = edge_index[0]
    dst = edge_index[1]
    h = _embed(x, Wn, bn.reshape(1, -1), blk=2000)
    e = _embed_edges(edge_attr, We, be.reshape(1, -1))
    for i in range(W1s.shape[0]):
        agg = _edge_aggregate(h, e, src, dst)
        scale_row = jnp.broadcast_to(1.0 + eps[i], (1, H))
        h = _layer_update(h, agg, W1s[i], b1s[i].reshape(1, -1),
                          W2s[i], b2s[i].reshape(1, -1), scale_row)
    out = _pool_mlp(h, batch.reshape(N // 2000, 1, 2000), Wm1,
                    bm1.reshape(1, -1), Wm2.reshape(1, -1),
                    bm2.reshape(1, 1))
    return out
